# baseline (speedup 1.0000x reference)
import functools

import jax
import jax.numpy as jnp
from jax import lax
from jax.experimental import pallas as pl
from jax.experimental.pallas import tpu as pltpu
from jax.experimental.pallas import tpu_sc as plsc

NC = 2
NS = 16
NW = NC * NS
L = 16


def _make_kernel(B, D):
    PW = B // NW
    C = 32
    NCH = PW // C

    mesh = plsc.VectorSubcoreMesh(
        core_axis_name="c", subcore_axis_name="s", num_cores=NC,
        num_subcores=NS)

    buf = lambda: pltpu.VMEM((C, D), jnp.float32)

    @functools.partial(
        pl.kernel,
        out_type=jax.ShapeDtypeStruct((B,), jnp.float32),
        mesh=mesh,
        compiler_params=pltpu.CompilerParams(needs_layout_passes=False),
        scratch_types=[
            pltpu.VMEM((PW,), jnp.int32),
            pltpu.VMEM((PW,), jnp.int32),
            pltpu.VMEM((PW,), jnp.int32),
            buf(), buf(), buf(), buf(), buf(), buf(),
            buf(), buf(), buf(), buf(), buf(), buf(),
            buf(), buf(), buf(), buf(), buf(), buf(),
            pltpu.VMEM((PW,), jnp.float32),
            pltpu.SemaphoreType.DMA,
            pltpu.SemaphoreType.DMA,
            pltpu.SemaphoreType.DMA,
        ],
    )
    def kern(hid_hbm, tid_hbm, rid_hbm, nre_hbm, nim_hbm, rre_hbm,
             rim_hbm, out_hbm,
             hidx, tidx, ridx,
             hre0, him0, tre0, tim0, rre0, rim0,
             hre1, him1, tre1, tim1, rre1, rim1,
             hre2, him2, tre2, tim2, rre2, rim2,
             out_v, sem0, sem1, sem2):
        cid = lax.axis_index("c")
        sid = lax.axis_index("s")
        wid = sid * NC + cid
        base = pl.multiple_of(wid * PW, PW)

        pltpu.sync_copy(hid_hbm.at[pl.ds(base, PW)], hidx)
        pltpu.sync_copy(tid_hbm.at[pl.ds(base, PW)], tidx)
        pltpu.sync_copy(rid_hbm.at[pl.ds(base, PW)], ridx)

        sets = [
            (hre0, him0, tre0, tim0, rre0, rim0),
            (hre1, him1, tre1, tim1, rre1, rim1),
            (hre2, him2, tre2, tim2, rre2, rim2),
        ]
        sems = [sem0, sem1, sem2]

        def fire(c):
            bufs = sets[c % 3]
            sem = sems[c % 3]
            hix = hidx.at[pl.ds(c * C, C)]
            tix = tidx.at[pl.ds(c * C, C)]
            rix = ridx.at[pl.ds(c * C, C)]
            return [
                pltpu.async_copy(nre_hbm.at[hix], bufs[0], sem),
                pltpu.async_copy(nim_hbm.at[hix], bufs[1], sem),
                pltpu.async_copy(nre_hbm.at[tix], bufs[2], sem),
                pltpu.async_copy(nim_hbm.at[tix], bufs[3], sem),
                pltpu.async_copy(rre_hbm.at[rix], bufs[4], sem),
                pltpu.async_copy(rim_hbm.at[rix], bufs[5], sem),
            ]

        def compute(c):
            hre, him, tre, tim, rre, rim = sets[c % 3]
            off = c * C
            lanes = lax.iota(jnp.int32, L)

            def group(g, _):
                def elem(e16, svec):
                    e = g * L + e16
                    acc = jnp.zeros((L,), jnp.float32)
                    for k in range(D // L):
                        sl = pl.ds(k * L, L)
                        hr = hre[e, sl]
                        hi = him[e, sl]
                        tr = tre[e, sl]
                        ti = tim[e, sl]
                        a = hr * tr + hi * ti
                        b = hr * ti - hi * tr
                        acc = acc + rre[e, sl] * a + rim[e, sl] * b
                    return jnp.where(lanes == e16, jnp.sum(acc), svec)

                svec = lax.fori_loop(0, L, elem, jnp.zeros((L,), jnp.float32))
                goff = pl.multiple_of(off + g * L, L)
                out_v[pl.ds(goff, L)] = svec
                return _

            lax.fori_loop(0, C // L, group, 0)

        ring = [fire(0), fire(1)]
        for c in range(NCH):
            if c + 2 < NCH:
                ring.append(fire(c + 2))
            for cp in ring.pop(0):
                cp.wait()
            compute(c)

        pltpu.sync_copy(out_v, out_hbm.at[pl.ds(base, PW)])

    return kern


def kernel(head_indices, tail_indices, relation_indices, node_real,
           node_img, rel_real, rel_img):
    B = head_indices.shape[0]
    D = node_real.shape[1]
    kern = _make_kernel(B, D)
    return kern(head_indices.astype(jnp.int32),
                tail_indices.astype(jnp.int32),
                relation_indices.astype(jnp.int32),
                node_real, node_img, rel_real, rel_img)


# speedup vs baseline: 1.0275x; 1.0275x over previous
import functools

import jax
import jax.numpy as jnp
from jax import lax
from jax.experimental import pallas as pl
from jax.experimental.pallas import tpu as pltpu
from jax.experimental.pallas import tpu_sc as plsc

NC = 2
NS = 16
NW = NC * NS
L = 16


def _make_kernel(B, D):
    PW = B // NW
    C = 64
    NCH = PW // C

    mesh = plsc.VectorSubcoreMesh(
        core_axis_name="c", subcore_axis_name="s", num_cores=NC,
        num_subcores=NS)

    buf = lambda: pltpu.VMEM((C, D), jnp.float32)

    @functools.partial(
        pl.kernel,
        out_type=jax.ShapeDtypeStruct((B,), jnp.float32),
        mesh=mesh,
        compiler_params=pltpu.CompilerParams(needs_layout_passes=False),
        scratch_types=[
            pltpu.VMEM((PW,), jnp.int32),
            pltpu.VMEM((PW,), jnp.int32),
            pltpu.VMEM((PW,), jnp.int32),
            buf(), buf(), buf(), buf(), buf(), buf(),
            buf(), buf(), buf(), buf(), buf(), buf(),
            pltpu.VMEM((PW,), jnp.float32),
            pltpu.SemaphoreType.DMA,
            pltpu.SemaphoreType.DMA,
        ],
    )
    def kern(hid_hbm, tid_hbm, rid_hbm, nre_hbm, nim_hbm, rre_hbm,
             rim_hbm, out_hbm,
             hidx, tidx, ridx,
             hre0, him0, tre0, tim0, rre0, rim0,
             hre1, him1, tre1, tim1, rre1, rim1,
             out_v, sem0, sem1):
        cid = lax.axis_index("c")
        sid = lax.axis_index("s")
        wid = sid * NC + cid
        base = pl.multiple_of(wid * PW, PW)

        pltpu.sync_copy(hid_hbm.at[pl.ds(base, PW)], hidx)
        pltpu.sync_copy(tid_hbm.at[pl.ds(base, PW)], tidx)
        pltpu.sync_copy(rid_hbm.at[pl.ds(base, PW)], ridx)

        sets = [
            (hre0, him0, tre0, tim0, rre0, rim0),
            (hre1, him1, tre1, tim1, rre1, rim1),
        ]
        sems = [sem0, sem1]

        def fire(c):
            bufs = sets[c % 2]
            sem = sems[c % 2]
            hix = hidx.at[pl.ds(c * C, C)]
            tix = tidx.at[pl.ds(c * C, C)]
            rix = ridx.at[pl.ds(c * C, C)]
            return [
                pltpu.async_copy(nre_hbm.at[hix], bufs[0], sem),
                pltpu.async_copy(nim_hbm.at[hix], bufs[1], sem),
                pltpu.async_copy(nre_hbm.at[tix], bufs[2], sem),
                pltpu.async_copy(nim_hbm.at[tix], bufs[3], sem),
                pltpu.async_copy(rre_hbm.at[rix], bufs[4], sem),
                pltpu.async_copy(rim_hbm.at[rix], bufs[5], sem),
            ]

        def compute(c):
            hre, him, tre, tim, rre, rim = sets[c % 2]
            off = c * C
            lanes = lax.iota(jnp.int32, L)

            def group(g, _):
                def elem(e16, svec):
                    e = g * L + e16
                    acc = jnp.zeros((L,), jnp.float32)
                    for k in range(D // L):
                        sl = pl.ds(k * L, L)
                        hr = hre[e, sl]
                        hi = him[e, sl]
                        tr = tre[e, sl]
                        ti = tim[e, sl]
                        a = hr * tr + hi * ti
                        b = hr * ti - hi * tr
                        acc = acc + rre[e, sl] * a + rim[e, sl] * b
                    return jnp.where(lanes == e16, jnp.sum(acc), svec)

                svec = lax.fori_loop(0, L, elem, jnp.zeros((L,), jnp.float32))
                goff = pl.multiple_of(off + g * L, L)
                out_v[pl.ds(goff, L)] = svec
                return _

            lax.fori_loop(0, C // L, group, 0)

        inflight = fire(0)
        for c in range(NCH):
            if c + 1 < NCH:
                nxt = fire(c + 1)
            for cp in inflight:
                cp.wait()
            compute(c)
            if c + 1 < NCH:
                inflight = nxt

        pltpu.sync_copy(out_v, out_hbm.at[pl.ds(base, PW)])

    return kern


def kernel(head_indices, tail_indices, relation_indices, node_real,
           node_img, rel_real, rel_img):
    B = head_indices.shape[0]
    D = node_real.shape[1]
    kern = _make_kernel(B, D)
    return kern(head_indices.astype(jnp.int32),
                tail_indices.astype(jnp.int32),
                relation_indices.astype(jnp.int32),
                node_real, node_img, rel_real, rel_img)


# speedup vs baseline: 1.0475x; 1.0195x over previous
import functools

import jax
import jax.numpy as jnp
from jax import lax
from jax.experimental import pallas as pl
from jax.experimental.pallas import tpu as pltpu
from jax.experimental.pallas import tpu_sc as plsc

NC = 2
NS = 16
NW = NC * NS
L = 16


def _make_kernel(B, D):
    PW = B // NW
    C = 64
    NCH = PW // C

    mesh = plsc.VectorSubcoreMesh(
        core_axis_name="c", subcore_axis_name="s", num_cores=NC,
        num_subcores=NS)

    buf = lambda: pltpu.VMEM((C, D), jnp.float32)

    @functools.partial(
        pl.kernel,
        out_type=jax.ShapeDtypeStruct((B,), jnp.float32),
        mesh=mesh,
        compiler_params=pltpu.CompilerParams(needs_layout_passes=False),
        scratch_types=[
            pltpu.VMEM((PW,), jnp.int32),
            pltpu.VMEM((PW,), jnp.int32),
            pltpu.VMEM((PW,), jnp.int32),
            buf(), buf(), buf(), buf(), buf(), buf(),
            buf(), buf(), buf(), buf(), buf(), buf(),
            pltpu.VMEM((PW,), jnp.float32),
            pltpu.SemaphoreType.DMA,
            pltpu.SemaphoreType.DMA,
        ],
    )
    def kern(hid_hbm, tid_hbm, rid_hbm, nre_hbm, nim_hbm, rre_hbm,
             rim_hbm, out_hbm,
             hidx, tidx, ridx,
             hre0, him0, tre0, tim0, rre0, rim0,
             hre1, him1, tre1, tim1, rre1, rim1,
             out_v, sem0, sem1):
        cid = lax.axis_index("c")
        sid = lax.axis_index("s")
        wid = sid * NC + cid
        base = pl.multiple_of(wid * PW, PW)

        idx_cps = [
            pltpu.async_copy(hid_hbm.at[pl.ds(base, PW)], hidx, sem0),
            pltpu.async_copy(tid_hbm.at[pl.ds(base, PW)], tidx, sem0),
            pltpu.async_copy(rid_hbm.at[pl.ds(base, PW)], ridx, sem0),
        ]
        for cp in idx_cps:
            cp.wait()

        sets = [
            (hre0, him0, tre0, tim0, rre0, rim0),
            (hre1, him1, tre1, tim1, rre1, rim1),
        ]
        sems = [sem0, sem1]

        def fire(c):
            bufs = sets[c % 2]
            sem = sems[c % 2]
            hix = hidx.at[pl.ds(c * C, C)]
            tix = tidx.at[pl.ds(c * C, C)]
            rix = ridx.at[pl.ds(c * C, C)]
            return [
                pltpu.async_copy(nre_hbm.at[hix], bufs[0], sem),
                pltpu.async_copy(nim_hbm.at[hix], bufs[1], sem),
                pltpu.async_copy(nre_hbm.at[tix], bufs[2], sem),
                pltpu.async_copy(nim_hbm.at[tix], bufs[3], sem),
                pltpu.async_copy(rre_hbm.at[rix], bufs[4], sem),
                pltpu.async_copy(rim_hbm.at[rix], bufs[5], sem),
            ]

        def compute(c):
            hre, him, tre, tim, rre, rim = sets[c % 2]
            off = c * C
            lanes = lax.iota(jnp.int32, L)

            def group(g, _):
                def elem(e16, svec):
                    e = g * L + e16
                    acc = jnp.zeros((L,), jnp.float32)
                    for k in range(D // L):
                        sl = pl.ds(k * L, L)
                        hr = hre[e, sl]
                        hi = him[e, sl]
                        tr = tre[e, sl]
                        ti = tim[e, sl]
                        a = hr * tr + hi * ti
                        b = hr * ti - hi * tr
                        acc = acc + rre[e, sl] * a + rim[e, sl] * b
                    return jnp.where(lanes == e16, jnp.sum(acc), svec)

                svec = lax.fori_loop(0, L, elem, jnp.zeros((L,), jnp.float32))
                goff = pl.multiple_of(off + g * L, L)
                out_v[pl.ds(goff, L)] = svec
                return _

            lax.fori_loop(0, C // L, group, 0)

        inflight = fire(0)
        for c in range(NCH):
            if c + 1 < NCH:
                nxt = fire(c + 1)
            for cp in inflight:
                cp.wait()
            compute(c)
            if c + 1 < NCH:
                inflight = nxt

        pltpu.sync_copy(out_v, out_hbm.at[pl.ds(base, PW)])

    return kern


def kernel(head_indices, tail_indices, relation_indices, node_real,
           node_img, rel_real, rel_img):
    B = head_indices.shape[0]
    D = node_real.shape[1]
    kern = _make_kernel(B, D)
    return kern(head_indices.astype(jnp.int32),
                tail_indices.astype(jnp.int32),
                relation_indices.astype(jnp.int32),
                node_real, node_img, rel_real, rel_img)


# speedup vs baseline: 1.0542x; 1.0064x over previous
import functools

import jax
import jax.numpy as jnp
from jax import lax
from jax.experimental import pallas as pl
from jax.experimental.pallas import tpu as pltpu
from jax.experimental.pallas import tpu_sc as plsc

NC = 2
NS = 16
NW = NC * NS
L = 16


def _make_kernel(B, D):
    PW = B // NW
    C = 64
    NCH = PW // C

    mesh = plsc.VectorSubcoreMesh(
        core_axis_name="c", subcore_axis_name="s", num_cores=NC,
        num_subcores=NS)

    buf = lambda: pltpu.VMEM((C, D), jnp.float32)

    @functools.partial(
        pl.kernel,
        out_type=jax.ShapeDtypeStruct((B,), jnp.float32),
        mesh=mesh,
        compiler_params=pltpu.CompilerParams(
            needs_layout_passes=False, skip_device_barrier=True),
        scratch_types=[
            pltpu.VMEM((PW,), jnp.int32),
            pltpu.VMEM((PW,), jnp.int32),
            pltpu.VMEM((PW,), jnp.int32),
            buf(), buf(), buf(), buf(), buf(), buf(),
            buf(), buf(), buf(), buf(), buf(), buf(),
            pltpu.VMEM((PW,), jnp.float32),
            pltpu.SemaphoreType.DMA,
            pltpu.SemaphoreType.DMA,
        ],
    )
    def kern(hid_hbm, tid_hbm, rid_hbm, nre_hbm, nim_hbm, rre_hbm,
             rim_hbm, out_hbm,
             hidx, tidx, ridx,
             hre0, him0, tre0, tim0, rre0, rim0,
             hre1, him1, tre1, tim1, rre1, rim1,
             out_v, sem0, sem1):
        cid = lax.axis_index("c")
        sid = lax.axis_index("s")
        wid = sid * NC + cid
        base = pl.multiple_of(wid * PW, PW)

        idx_cps = [
            pltpu.async_copy(hid_hbm.at[pl.ds(base, PW)], hidx, sem0),
            pltpu.async_copy(tid_hbm.at[pl.ds(base, PW)], tidx, sem0),
            pltpu.async_copy(rid_hbm.at[pl.ds(base, PW)], ridx, sem0),
        ]
        for cp in idx_cps:
            cp.wait()

        sets = [
            (hre0, him0, tre0, tim0, rre0, rim0),
            (hre1, him1, tre1, tim1, rre1, rim1),
        ]
        sems = [sem0, sem1]

        def fire(c):
            bufs = sets[c % 2]
            sem = sems[c % 2]
            hix = hidx.at[pl.ds(c * C, C)]
            tix = tidx.at[pl.ds(c * C, C)]
            rix = ridx.at[pl.ds(c * C, C)]
            return [
                pltpu.async_copy(nre_hbm.at[hix], bufs[0], sem),
                pltpu.async_copy(nim_hbm.at[hix], bufs[1], sem),
                pltpu.async_copy(nre_hbm.at[tix], bufs[2], sem),
                pltpu.async_copy(nim_hbm.at[tix], bufs[3], sem),
                pltpu.async_copy(rre_hbm.at[rix], bufs[4], sem),
                pltpu.async_copy(rim_hbm.at[rix], bufs[5], sem),
            ]

        def compute(c):
            hre, him, tre, tim, rre, rim = sets[c % 2]
            off = c * C
            lanes = lax.iota(jnp.int32, L)

            def group(g, _):
                def elem(e16, svec):
                    e = g * L + e16
                    acc = jnp.zeros((L,), jnp.float32)
                    for k in range(D // L):
                        sl = pl.ds(k * L, L)
                        hr = hre[e, sl]
                        hi = him[e, sl]
                        tr = tre[e, sl]
                        ti = tim[e, sl]
                        a = hr * tr + hi * ti
                        b = hr * ti - hi * tr
                        acc = acc + rre[e, sl] * a + rim[e, sl] * b
                    return jnp.where(lanes == e16, jnp.sum(acc), svec)

                svec = lax.fori_loop(0, L, elem, jnp.zeros((L,), jnp.float32))
                goff = pl.multiple_of(off + g * L, L)
                out_v[pl.ds(goff, L)] = svec
                return _

            lax.fori_loop(0, C // L, group, 0)

        inflight = fire(0)
        for c in range(NCH):
            if c + 1 < NCH:
                nxt = fire(c + 1)
            for cp in inflight:
                cp.wait()
            compute(c)
            if c + 1 < NCH:
                inflight = nxt

        pltpu.sync_copy(out_v, out_hbm.at[pl.ds(base, PW)])

    return kern


def kernel(head_indices, tail_indices, relation_indices, node_real,
           node_img, rel_real, rel_img):
    B = head_indices.shape[0]
    D = node_real.shape[1]
    kern = _make_kernel(B, D)
    return kern(head_indices.astype(jnp.int32),
                tail_indices.astype(jnp.int32),
                relation_indices.astype(jnp.int32),
                node_real, node_img, rel_real, rel_img)
